# baseline (speedup 1.0000x reference)
import functools

import jax
import jax.numpy as jnp
from jax.experimental import pallas as pl
from jax.experimental.pallas import tpu as pltpu


def _round_up(x, m):
    return (x + m - 1) // m * m


def _fused_rows_kernel(x_ref, wi0_ref, b0_ref, wir_ref, br_ref, wo_ref,
                       bo_ref, out_ref, lp_ref, *, num_rest, seq):
    cdt = wi0_ref.dtype
    h = jnp.tanh(
        jnp.dot(x_ref[...], wi0_ref[...],
                preferred_element_type=jnp.float32) + b0_ref[...])
    for j in range(num_rest):
        h = jnp.tanh(
            jnp.dot(h.astype(cdt), wir_ref[j],
                    preferred_element_type=jnp.float32) + br_ref[j])
    out_ref[...] = h

    tm, hp = h.shape
    nb = tm // seq
    last = h.reshape(nb, seq, hp)[:, seq - 1, :]
    logits = jnp.dot(last.astype(wo_ref.dtype), wo_ref[...],
                     preferred_element_type=jnp.float32) + bo_ref[...]
    m = jnp.max(logits, axis=-1, keepdims=True)
    e = logits - m
    lp_ref[...] = e - jnp.log(jnp.sum(jnp.exp(e), axis=-1, keepdims=True))


def _head_kernel(h_ref, wo_ref, bo_ref, lp_ref):
    last = h_ref[:, 0, :]
    logits = jnp.dot(last.astype(wo_ref.dtype), wo_ref[...],
                     preferred_element_type=jnp.float32) + bo_ref[...]
    m = jnp.max(logits, axis=-1, keepdims=True)
    e = logits - m
    lp_ref[...] = e - jnp.log(jnp.sum(jnp.exp(e), axis=-1, keepdims=True))


@jax.jit
def _forward(x, wi0, bi0, wir, bir, bh, wo, bo):
    batch, seq, d_in = x.shape
    hidden = wi0.shape[-1]
    l_rest = wir.shape[0]

    b0 = bi0 + bh[0]
    br = bir + bh[1:] if l_rest > 0 else jnp.zeros((1, 1, hidden), jnp.float32)
    wir_e = wir if l_rest > 0 else jnp.zeros((1, hidden, hidden), jnp.float32)
    l_eff = wir_e.shape[0]

    hp = _round_up(hidden, 128)
    if hp != hidden:
        wi0 = jnp.pad(wi0, ((0, 0), (0, hp - hidden)))
        b0 = jnp.pad(b0, ((0, 0), (0, hp - hidden)))
        wir_e = jnp.pad(wir_e, ((0, 0), (0, hp - hidden), (0, hp - hidden)))
        br = jnp.pad(br, ((0, 0), (0, 0), (0, hp - hidden)))
        wo_p = jnp.pad(wo, ((0, hp - hidden), (0, 0)))
    else:
        wo_p = wo
    out_size = wo.shape[-1]
    op = _round_up(out_size, 128)
    if op != out_size:
        wo_p = jnp.pad(wo_p, ((0, 0), (0, op - out_size)),
                       constant_values=0.0)
        bo_p = jnp.pad(bo, ((0, 0), (0, op - out_size)),
                       constant_values=-jnp.inf)
    else:
        bo_p = bo

    cdt = jnp.bfloat16
    x_rows = x.reshape(batch * seq, d_in).astype(cdt)
    wi0_c = wi0.astype(cdt)
    wir_c = wir_e.astype(cdt)
    wo_c = wo_p.astype(cdt)

    rows = batch * seq
    nb = max(1, min(batch, 512 // seq if seq <= 512 else 1))
    while batch % nb != 0:
        nb -= 1
    tm = nb * seq
    fuse_head = (rows % tm == 0) and (tm % seq == 0)

    grid = (rows // tm,) if fuse_head else (pl.cdiv(rows, tm),)

    def w_spec(shape, index_map):
        return pl.BlockSpec(shape, index_map, pipeline_mode=pl.Buffered(1))

    vmem_limit = int(min(128 * 1024 * 1024, 2 * (
        2 * tm * d_in * 2 + 2 * tm * hp * 4 + 2 * nb * op * 4
        + d_in * hp * 2 + l_eff * hp * hp * 2 + hp * op * 2
        + (1 + l_eff) * hp * 4 + op * 4)))
    cost = pl.CostEstimate(
        flops=2 * rows * (d_in + l_rest * hp) * hp + 2 * batch * hp * op,
        transcendentals=rows * hp * (1 + l_rest) + batch * op,
        bytes_accessed=(rows * d_in * 2 + rows * hp * 4 + batch * op * 4
                        + d_in * hp * 2 + l_eff * hp * hp * 2 + hp * op * 2))

    if fuse_head:
        kfn = functools.partial(_fused_rows_kernel, num_rest=l_rest, seq=seq)
        h_rows, lp = pl.pallas_call(
            kfn,
            out_shape=(jax.ShapeDtypeStruct((rows, hp), jnp.float32),
                       jax.ShapeDtypeStruct((batch, op), jnp.float32)),
            grid=grid,
            in_specs=[
                pl.BlockSpec((tm, d_in), lambda i: (i, 0)),
                w_spec((d_in, hp), lambda i: (0, 0)),
                w_spec((1, hp), lambda i: (0, 0)),
                w_spec((l_eff, hp, hp), lambda i: (0, 0, 0)),
                w_spec((l_eff, 1, hp), lambda i: (0, 0, 0)),
                w_spec((hp, op), lambda i: (0, 0)),
                w_spec((1, op), lambda i: (0, 0)),
            ],
            out_specs=(pl.BlockSpec((tm, hp), lambda i: (i, 0)),
                       pl.BlockSpec((nb, op), lambda i: (i, 0))),
            compiler_params=pltpu.CompilerParams(
                dimension_semantics=("parallel",),
                vmem_limit_bytes=vmem_limit),
            cost_estimate=cost,
        )(x_rows, wi0_c, b0, wir_c, br, wo_c, bo_p)
    else:
        kfn = functools.partial(_rows_only_kernel, num_rest=l_rest)
        h_rows = pl.pallas_call(
            kfn,
            out_shape=jax.ShapeDtypeStruct((rows, hp), jnp.float32),
            grid=grid,
            in_specs=[
                pl.BlockSpec((tm, d_in), lambda i: (i, 0)),
                w_spec((d_in, hp), lambda i: (0, 0)),
                w_spec((1, hp), lambda i: (0, 0)),
                w_spec((l_eff, hp, hp), lambda i: (0, 0, 0)),
                w_spec((l_eff, 1, hp), lambda i: (0, 0, 0)),
            ],
            out_specs=pl.BlockSpec((tm, hp), lambda i: (i, 0)),
            compiler_params=pltpu.CompilerParams(
                dimension_semantics=("parallel",),
                vmem_limit_bytes=vmem_limit),
            cost_estimate=cost,
        )(x_rows, wi0_c, b0, wir_c, br)
        h3 = h_rows.reshape(batch, seq, hp)
        lp = pl.pallas_call(
            _head_kernel,
            out_shape=jax.ShapeDtypeStruct((batch, op), jnp.float32),
            grid=(1,),
            in_specs=[
                pl.BlockSpec((batch, 1, hp), lambda i: (0, seq - 1, 0)),
                pl.BlockSpec((hp, op), lambda i: (0, 0)),
                pl.BlockSpec((1, op), lambda i: (0, 0)),
            ],
            out_specs=pl.BlockSpec((batch, op), lambda i: (0, 0)),
        )(h3, wo_c, bo_p)

    out3 = h_rows.reshape(batch, seq, hp)
    outputs = out3[..., :hidden] if hp != hidden else out3
    log_probs = lp[:, :out_size] if op != out_size else lp
    return log_probs, outputs


def _rows_only_kernel(x_ref, wi0_ref, b0_ref, wir_ref, br_ref, out_ref, *,
                      num_rest):
    cdt = wi0_ref.dtype
    h = jnp.tanh(
        jnp.dot(x_ref[...], wi0_ref[...],
                preferred_element_type=jnp.float32) + b0_ref[...])
    for j in range(num_rest):
        h = jnp.tanh(
            jnp.dot(h.astype(cdt), wir_ref[j],
                    preferred_element_type=jnp.float32) + br_ref[j])
    out_ref[...] = h


def kernel(x, wi0, bi0, wir, bir, wh, bh, wo, bo):
    return _forward(x, wi0, bi0, wir, bir, bh, wo, bo)
